# baseline (speedup 1.0000x reference)
import functools

import jax
import jax.numpy as jnp
from jax import lax
from jax.experimental import pallas as pl
from jax.experimental.pallas import tpu as pltpu
from jax.experimental.pallas import tpu_sc as plsc

_MARGIN = 5.0
_NW = 32
_LANES = 16
_B = 1024
_ROWL = 2 * _B + 1
_CH = 4096
_UNROLL = 16


def _stage_scratch():
    return [
        pltpu.VMEM((_CH,), jnp.float32),
        pltpu.VMEM((_CH,), jnp.int32),
        pltpu.VMEM((_CH,), jnp.float32),
        pltpu.VMEM((_CH,), jnp.int32),
        pltpu.SemaphoreType.DMA,
        pltpu.SemaphoreType.DMA,
    ]


def _zero_ref(ref, nwords, dtype):
    z = jnp.zeros((_LANES,), dtype)
    nsteps = nwords // _LANES

    def zero_body(t, _):
        for u in range(_UNROLL):
            ref[pl.ds((t * _UNROLL + u) * _LANES, _LANES)] = z
        return 0

    lax.fori_loop(0, nsteps // _UNROLL, zero_body, 0)
    for t in range((nsteps // _UNROLL) * _UNROLL, nsteps):
        ref[pl.ds(t * _LANES, _LANES)] = z


def _scan_chunks(logits_hbm, labels_hbm, base, nchunk, bufs, per_vector):
    def start(c, buf):
        lv, bv, sm = buf
        off = base + c * _CH
        pltpu.async_copy(logits_hbm.at[pl.ds(off, _CH)], lv, sm)
        pltpu.async_copy(labels_hbm.at[pl.ds(off, _CH)], bv, sm)

    def wait(buf):
        lv, bv, sm = buf
        pltpu.make_async_copy(logits_hbm.at[pl.ds(base, _CH)], lv, sm).wait()
        pltpu.make_async_copy(labels_hbm.at[pl.ds(base, _CH)], bv, sm).wait()

    def process(lv, bv):
        @plsc.parallel_loop(0, _CH // _LANES, unroll=_UNROLL)
        def _(i):
            o = i * _LANES
            per_vector(lv[pl.ds(o, _LANES)], bv[pl.ds(o, _LANES)])

    start(0, bufs[0])
    start(1, bufs[1])

    def chunk_body(g, _):
        for b in range(2):
            c = g * 2 + b
            buf = bufs[b]
            wait(buf)
            process(buf[0], buf[1])

            @pl.when(c + 2 < nchunk)
            def _():
                start(c + 2, buf)

        return 0

    lax.fori_loop(0, nchunk // 2, chunk_body, 0)


def _merge_lanes(src, dst, dtype):
    for lab in range(2):
        def merge_body(g, _, lab=lab):
            col = lab * _B + g * _LANES
            acc = jnp.zeros((_LANES,), dtype)
            for l in range(_LANES):
                acc = acc + src[pl.ds(l * _ROWL + col, _LANES)]
            dst[lab, pl.ds(g * _LANES, _LANES)] = acc
            return 0

        lax.fori_loop(0, _B // _LANES, merge_body, 0)


def _keys(x, lb):
    iu = lax.bitcast_convert_type(x, jnp.int32)
    k0 = iu ^ ((iu >> 31) | jnp.int32(-2147483648))
    return k0 ^ (lb - 1)


def _make_sc_pass1(interpret=False):
    mesh = plsc.VectorSubcoreMesh(
        core_axis_name="c", subcore_axis_name="s", num_cores=2, num_subcores=16
    )
    out_type = jax.ShapeDtypeStruct((_NW, 2, _B), jnp.int32)
    scratch = _stage_scratch() + [
        pltpu.VMEM((_LANES * _ROWL,), jnp.int32),
        pltpu.VMEM((2, _B), jnp.int32),
    ]

    def body(logits_hbm, labels_hbm, cnt_out,
             logv0, labv0, logv1, labv1, sem0, sem1, cnth, mc):
        wid = lax.axis_index("s") * 2 + lax.axis_index("c")
        n_per_w = logits_hbm.shape[0] // _NW
        base = wid * n_per_w

        _zero_ref(cnth, _LANES * _ROWL, jnp.int32)

        lane = lax.broadcasted_iota(jnp.int32, (_LANES,), 0)
        lane_base = lane * _ROWL
        ones = jnp.ones((_LANES,), jnp.int32)

        def per_vector(x, lb):
            key = _keys(x, lb)
            bucket = lax.shift_right_logical(key, 22)
            flat = lane_base + ((lb << 10) | bucket)
            plsc.addupdate_scatter(cnth, [flat], ones)

        bufs = ((logv0, labv0, sem0), (logv1, labv1, sem1))
        _scan_chunks(logits_hbm, labels_hbm, base, n_per_w // _CH, bufs,
                     per_vector)

        _merge_lanes(cnth, mc, jnp.int32)
        pltpu.sync_copy(mc, cnt_out.at[wid])

    return pl.kernel(
        body,
        out_type=out_type,
        mesh=mesh,
        scratch_types=scratch,
        compiler_params=pltpu.CompilerParams(needs_layout_passes=False),
        interpret=interpret,
    )


def _make_sc_pass2(interpret=False):
    mesh = plsc.VectorSubcoreMesh(
        core_axis_name="c", subcore_axis_name="s", num_cores=2, num_subcores=16
    )
    out_type = (
        jax.ShapeDtypeStruct((_NW, 2, _B), jnp.int32),
        jax.ShapeDtypeStruct((_NW, 2, _B), jnp.float32),
        jax.ShapeDtypeStruct((_NW, 2, _LANES), jnp.float32),
    )
    scratch = _stage_scratch() + [
        pltpu.VMEM((_LANES * _ROWL,), jnp.int32),
        pltpu.VMEM((_LANES * _ROWL,), jnp.float32),
        pltpu.VMEM((3 * _LANES,), jnp.float32),
        pltpu.VMEM((2, _B), jnp.int32),
        pltpu.VMEM((2, _B), jnp.float32),
        pltpu.VMEM((2, _LANES), jnp.float32),
        pltpu.VMEM((16,), jnp.int32),
        pltpu.VMEM((16,), jnp.int32),
    ]

    def body(logits_hbm, labels_hbm, prefix_hbm, cnt_out, sum_out, bel_out,
             logv0, labv0, logv1, labv1, sem0, sem1,
             cnth, sumh, belh, mc, ms, bs, p0v, p1v):
        wid = lax.axis_index("s") * 2 + lax.axis_index("c")
        n_per_w = logits_hbm.shape[0] // _NW
        base = wid * n_per_w

        _zero_ref(cnth, _LANES * _ROWL, jnp.int32)
        _zero_ref(sumh, _LANES * _ROWL, jnp.float32)
        _zero_ref(belh, 3 * _LANES, jnp.float32)

        pltpu.sync_copy(prefix_hbm.at[0], p0v)
        pltpu.sync_copy(prefix_hbm.at[1], p1v)
        sel0 = p0v[...]
        sel1 = p1v[...]

        lane = lax.broadcasted_iota(jnp.int32, (_LANES,), 0)
        lane_base = lane * _ROWL
        lane3 = lane * 3
        ones = jnp.ones((_LANES,), jnp.int32)

        def per_vector(x, lb):
            ispos = lb == 1
            key = _keys(x, lb)
            y = jnp.where(ispos, x, -x)
            pref = lax.shift_right_logical(key, 22)
            selv = jnp.where(ispos, sel1, sel0)
            d = pref - selv
            meq = d == 0
            mlt = d < 0
            bucket = lax.shift_right_logical(key, 12) & (_B - 1)
            flat = lane_base + ((lb << 10) | bucket)
            plsc.addupdate_scatter(cnth, [flat], ones, mask=meq)
            plsc.addupdate_scatter(sumh, [flat], y, mask=meq)
            plsc.addupdate_scatter(belh, [lane3 + lb], y, mask=mlt)

        bufs = ((logv0, labv0, sem0), (logv1, labv1, sem1))
        _scan_chunks(logits_hbm, labels_hbm, base, n_per_w // _CH, bufs,
                     per_vector)

        _merge_lanes(cnth, mc, jnp.int32)
        _merge_lanes(sumh, ms, jnp.float32)
        for lb in range(2):
            bs[lb, :] = plsc.load_gather(belh, [lane3 + lb])
        pltpu.sync_copy(mc, cnt_out.at[wid])
        pltpu.sync_copy(ms, sum_out.at[wid])
        pltpu.sync_copy(bs, bel_out.at[wid])

    return pl.kernel(
        body,
        out_type=out_type,
        mesh=mesh,
        scratch_types=scratch,
        compiler_params=pltpu.CompilerParams(needs_layout_passes=False),
        interpret=interpret,
    )


def _shift_right_cols(x, s):
    pad = jnp.zeros((x.shape[0], s), x.dtype)
    return jnp.concatenate([pad, x[:, :-s]], axis=1)


def _cumsum_cols(x):
    s = 1
    while s < _B:
        x = x + _shift_right_cols(x, s)
        s *= 2
    return x


def _make_tc_select(interpret=False):
    def body(cnt_ref, pr_ref, cb_ref, kk_ref, nn_ref):
        C = jnp.sum(cnt_ref[...], axis=0)
        n_c = jnp.sum(C, axis=1, keepdims=True)
        rowi = lax.broadcasted_iota(jnp.int32, (2, 1), 0)
        divisor = jnp.where(rowi == 1, 2, 10)
        k_c = jnp.maximum(1, n_c // divisor)
        cum = _cumsum_cols(C)
        ltm = cum < k_c
        b_idx = jnp.sum(ltm.astype(jnp.int32), axis=1, keepdims=True)
        cb = jnp.sum(jnp.where(ltm, C, 0), axis=1, keepdims=True)
        pr_ref[...] = jnp.broadcast_to(b_idx, (2, 16))
        cb_ref[...] = jnp.broadcast_to(cb, (2, 16))
        kk_ref[...] = jnp.broadcast_to(k_c, (2, 16))
        nn_ref[...] = jnp.broadcast_to(n_c, (2, 16))

    return pl.pallas_call(
        body,
        out_shape=(
            jax.ShapeDtypeStruct((2, 16), jnp.int32),
            jax.ShapeDtypeStruct((2, 16), jnp.int32),
            jax.ShapeDtypeStruct((2, 16), jnp.int32),
            jax.ShapeDtypeStruct((2, 16), jnp.int32),
        ),
        interpret=interpret,
    )


def _make_tc_final(interpret=False):
    def body(cnt_ref, sum_ref, bel_ref, cb_in, kk_in, nn_in, loss_ref):
        C = jnp.sum(cnt_ref[...], axis=0)
        S = jnp.sum(sum_ref[...], axis=0)
        sbel = jnp.sum(jnp.sum(bel_ref[...], axis=0), axis=1, keepdims=True)
        prev_cb = cb_in[:, :1]
        k_c = kk_in[:, :1]
        n_c = nn_in[:, :1]
        need = k_c - prev_cb
        cum = _cumsum_cols(C)
        ltm = cum < need
        cb2 = jnp.sum(jnp.where(ltm, C, 0), axis=1, keepdims=True)
        sb2 = jnp.sum(jnp.where(ltm, S, 0.0), axis=1, keepdims=True)
        eqm = jnp.logical_and(jnp.logical_not(ltm), (cum - C) < need)
        c_at = jnp.sum(jnp.where(eqm, C, 0), axis=1, keepdims=True)
        s_at = jnp.sum(jnp.where(eqm, S, 0.0), axis=1, keepdims=True)
        r = (need - cb2).astype(jnp.float32)
        est = sbel + sb2 + r * s_at / jnp.maximum(c_at, 1).astype(jnp.float32)
        means = est / k_c.astype(jnp.float32)
        diff = jnp.sum(means)
        loss = jnp.maximum(_MARGIN - diff, 0.0)
        empty = jnp.sum(jnp.where(n_c == 0, 1, 0)) > 0
        loss = jnp.where(empty, 0.0, loss)
        loss_ref[...] = jnp.broadcast_to(loss, (1, 1))

    return pl.pallas_call(
        body,
        out_shape=jax.ShapeDtypeStruct((1, 1), jnp.float32),
        interpret=interpret,
    )


_sc_pass1_cached = functools.cache(_make_sc_pass1)
_sc_pass2_cached = functools.cache(_make_sc_pass2)
_tc_sel = _make_tc_select()
_tc_fin = _make_tc_final()


@jax.jit
def kernel(logits, labels):
    assert logits.shape[0] % (_NW * _CH) == 0
    cnt1 = _sc_pass1_cached()(logits, labels)
    pr1, cb1, kk1, nn1 = _tc_sel(cnt1)
    cnt2, sum2, bel = _sc_pass2_cached()(logits, labels, pr1)
    loss = _tc_fin(cnt2, sum2, bel, cb1, kk1, nn1)
    return loss.reshape(())


# speedup vs baseline: 1.9185x; 1.9185x over previous
import functools

import jax
import jax.numpy as jnp
from jax import lax
from jax.experimental import pallas as pl
from jax.experimental.pallas import tpu as pltpu
from jax.experimental.pallas import tpu_sc as plsc

_MARGIN = 5.0
_NW = 32
_LANES = 16
_B = 1024
_ROWL = 2 * _B + 1
_CH = 4096
_UNROLL = 4


def _stage_scratch():
    return [
        pltpu.VMEM((_CH,), jnp.float32),
        pltpu.VMEM((_CH,), jnp.int32),
        pltpu.VMEM((_CH,), jnp.float32),
        pltpu.VMEM((_CH,), jnp.int32),
        pltpu.SemaphoreType.DMA,
        pltpu.SemaphoreType.DMA,
    ]


def _zero_ref(ref, nwords, dtype):
    z = jnp.zeros((_LANES,), dtype)
    nsteps = nwords // _LANES

    def zero_body(t, _):
        for u in range(_UNROLL):
            ref[pl.ds((t * _UNROLL + u) * _LANES, _LANES)] = z
        return 0

    lax.fori_loop(0, nsteps // _UNROLL, zero_body, 0)
    for t in range((nsteps // _UNROLL) * _UNROLL, nsteps):
        ref[pl.ds(t * _LANES, _LANES)] = z


def _scan_chunks(logits_hbm, labels_hbm, base, nchunk, bufs, per_vector):
    def start(c, buf):
        lv, bv, sm = buf
        off = base + c * _CH
        pltpu.async_copy(logits_hbm.at[pl.ds(off, _CH)], lv, sm)
        pltpu.async_copy(labels_hbm.at[pl.ds(off, _CH)], bv, sm)

    def wait(buf):
        lv, bv, sm = buf
        pltpu.make_async_copy(logits_hbm.at[pl.ds(base, _CH)], lv, sm).wait()
        pltpu.make_async_copy(labels_hbm.at[pl.ds(base, _CH)], bv, sm).wait()

    def process(lv, bv):
        @plsc.parallel_loop(0, _CH // _LANES, unroll=_UNROLL)
        def _(i):
            o = i * _LANES
            per_vector(lv[pl.ds(o, _LANES)], bv[pl.ds(o, _LANES)])

    start(0, bufs[0])
    start(1, bufs[1])

    def chunk_body(g, _):
        for b in range(2):
            c = g * 2 + b
            buf = bufs[b]
            wait(buf)
            process(buf[0], buf[1])

            @pl.when(c + 2 < nchunk)
            def _():
                start(c + 2, buf)

        return 0

    lax.fori_loop(0, nchunk // 2, chunk_body, 0)


def _merge_lanes(src, dst, dtype):
    for lab in range(2):
        def merge_body(g, _, lab=lab):
            col = lab * _B + g * _LANES
            acc = jnp.zeros((_LANES,), dtype)
            for l in range(_LANES):
                acc = acc + src[pl.ds(l * _ROWL + col, _LANES)]
            dst[lab, pl.ds(g * _LANES, _LANES)] = acc
            return 0

        lax.fori_loop(0, _B // _LANES, merge_body, 0)


def _keys(x, lb):
    iu = lax.bitcast_convert_type(x, jnp.int32)
    k0 = iu ^ ((iu >> 31) | jnp.int32(-2147483648))
    return k0 ^ (lb - 1)


def _make_sc_pass1(interpret=False):
    mesh = plsc.VectorSubcoreMesh(
        core_axis_name="c", subcore_axis_name="s", num_cores=2, num_subcores=16
    )
    out_type = jax.ShapeDtypeStruct((_NW, 2, _B), jnp.int32)
    scratch = _stage_scratch() + [
        pltpu.VMEM((_LANES * _ROWL,), jnp.int32),
        pltpu.VMEM((2, _B), jnp.int32),
    ]

    def body(logits_hbm, labels_hbm, cnt_out,
             logv0, labv0, logv1, labv1, sem0, sem1, cnth, mc):
        wid = lax.axis_index("s") * 2 + lax.axis_index("c")
        n_per_w = logits_hbm.shape[0] // _NW
        base = wid * n_per_w

        _zero_ref(cnth, _LANES * _ROWL, jnp.int32)

        lane = lax.broadcasted_iota(jnp.int32, (_LANES,), 0)
        lane_base = lane * _ROWL
        ones = jnp.ones((_LANES,), jnp.int32)

        def per_vector(x, lb):
            key = _keys(x, lb)
            bucket = lax.shift_right_logical(key, 22)
            flat = lane_base + ((lb << 10) | bucket)
            plsc.addupdate_scatter(cnth, [flat], ones)

        bufs = ((logv0, labv0, sem0), (logv1, labv1, sem1))
        _scan_chunks(logits_hbm, labels_hbm, base, n_per_w // _CH, bufs,
                     per_vector)

        _merge_lanes(cnth, mc, jnp.int32)
        pltpu.sync_copy(mc, cnt_out.at[wid])

    return pl.kernel(
        body,
        out_type=out_type,
        mesh=mesh,
        scratch_types=scratch,
        compiler_params=pltpu.CompilerParams(needs_layout_passes=False),
        interpret=interpret,
    )


def _make_sc_pass2(interpret=False):
    mesh = plsc.VectorSubcoreMesh(
        core_axis_name="c", subcore_axis_name="s", num_cores=2, num_subcores=16
    )
    out_type = (
        jax.ShapeDtypeStruct((_NW, 2, _B), jnp.int32),
        jax.ShapeDtypeStruct((_NW, 2, _B), jnp.float32),
        jax.ShapeDtypeStruct((_NW, 2, _LANES), jnp.float32),
    )
    scratch = _stage_scratch() + [
        pltpu.VMEM((_LANES * _ROWL,), jnp.int32),
        pltpu.VMEM((_LANES * _ROWL,), jnp.float32),
        pltpu.VMEM((3 * _LANES,), jnp.float32),
        pltpu.VMEM((2, _B), jnp.int32),
        pltpu.VMEM((2, _B), jnp.float32),
        pltpu.VMEM((2, _LANES), jnp.float32),
        pltpu.VMEM((16,), jnp.int32),
        pltpu.VMEM((16,), jnp.int32),
    ]

    def body(logits_hbm, labels_hbm, prefix_hbm, cnt_out, sum_out, bel_out,
             logv0, labv0, logv1, labv1, sem0, sem1,
             cnth, sumh, belh, mc, ms, bs, p0v, p1v):
        wid = lax.axis_index("s") * 2 + lax.axis_index("c")
        n_per_w = logits_hbm.shape[0] // _NW
        base = wid * n_per_w

        _zero_ref(cnth, _LANES * _ROWL, jnp.int32)
        _zero_ref(sumh, _LANES * _ROWL, jnp.float32)
        _zero_ref(belh, 3 * _LANES, jnp.float32)

        pltpu.sync_copy(prefix_hbm.at[0], p0v)
        pltpu.sync_copy(prefix_hbm.at[1], p1v)
        sel0 = p0v[...]
        sel1 = p1v[...]

        lane = lax.broadcasted_iota(jnp.int32, (_LANES,), 0)
        lane_base = lane * _ROWL
        lane3 = lane * 3
        ones = jnp.ones((_LANES,), jnp.int32)

        def per_vector(x, lb):
            ispos = lb == 1
            key = _keys(x, lb)
            y = jnp.where(ispos, x, -x)
            pref = lax.shift_right_logical(key, 22)
            selv = jnp.where(ispos, sel1, sel0)
            d = pref - selv
            meq = d == 0
            mlt = d < 0
            bucket = lax.shift_right_logical(key, 12) & (_B - 1)
            flat = lane_base + ((lb << 10) | bucket)
            plsc.addupdate_scatter(cnth, [flat], ones, mask=meq)
            plsc.addupdate_scatter(sumh, [flat], y, mask=meq)
            plsc.addupdate_scatter(belh, [lane3 + lb], y, mask=mlt)

        bufs = ((logv0, labv0, sem0), (logv1, labv1, sem1))
        _scan_chunks(logits_hbm, labels_hbm, base, n_per_w // _CH, bufs,
                     per_vector)

        _merge_lanes(cnth, mc, jnp.int32)
        _merge_lanes(sumh, ms, jnp.float32)
        for lb in range(2):
            bs[lb, :] = plsc.load_gather(belh, [lane3 + lb])
        pltpu.sync_copy(mc, cnt_out.at[wid])
        pltpu.sync_copy(ms, sum_out.at[wid])
        pltpu.sync_copy(bs, bel_out.at[wid])

    return pl.kernel(
        body,
        out_type=out_type,
        mesh=mesh,
        scratch_types=scratch,
        compiler_params=pltpu.CompilerParams(needs_layout_passes=False),
        interpret=interpret,
    )


def _shift_right_cols(x, s):
    pad = jnp.zeros((x.shape[0], s), x.dtype)
    return jnp.concatenate([pad, x[:, :-s]], axis=1)


def _cumsum_cols(x):
    s = 1
    while s < _B:
        x = x + _shift_right_cols(x, s)
        s *= 2
    return x


def _make_tc_select(interpret=False):
    def body(cnt_ref, pr_ref, cb_ref, kk_ref, nn_ref):
        C = jnp.sum(cnt_ref[...], axis=0)
        n_c = jnp.sum(C, axis=1, keepdims=True)
        rowi = lax.broadcasted_iota(jnp.int32, (2, 1), 0)
        divisor = jnp.where(rowi == 1, 2, 10)
        k_c = jnp.maximum(1, n_c // divisor)
        cum = _cumsum_cols(C)
        ltm = cum < k_c
        b_idx = jnp.sum(ltm.astype(jnp.int32), axis=1, keepdims=True)
        cb = jnp.sum(jnp.where(ltm, C, 0), axis=1, keepdims=True)
        pr_ref[...] = jnp.broadcast_to(b_idx, (2, 16))
        cb_ref[...] = jnp.broadcast_to(cb, (2, 16))
        kk_ref[...] = jnp.broadcast_to(k_c, (2, 16))
        nn_ref[...] = jnp.broadcast_to(n_c, (2, 16))

    return pl.pallas_call(
        body,
        out_shape=(
            jax.ShapeDtypeStruct((2, 16), jnp.int32),
            jax.ShapeDtypeStruct((2, 16), jnp.int32),
            jax.ShapeDtypeStruct((2, 16), jnp.int32),
            jax.ShapeDtypeStruct((2, 16), jnp.int32),
        ),
        interpret=interpret,
    )


def _make_tc_final(interpret=False):
    def body(cnt_ref, sum_ref, bel_ref, cb_in, kk_in, nn_in, loss_ref):
        C = jnp.sum(cnt_ref[...], axis=0)
        S = jnp.sum(sum_ref[...], axis=0)
        sbel = jnp.sum(jnp.sum(bel_ref[...], axis=0), axis=1, keepdims=True)
        prev_cb = cb_in[:, :1]
        k_c = kk_in[:, :1]
        n_c = nn_in[:, :1]
        need = k_c - prev_cb
        cum = _cumsum_cols(C)
        ltm = cum < need
        cb2 = jnp.sum(jnp.where(ltm, C, 0), axis=1, keepdims=True)
        sb2 = jnp.sum(jnp.where(ltm, S, 0.0), axis=1, keepdims=True)
        eqm = jnp.logical_and(jnp.logical_not(ltm), (cum - C) < need)
        c_at = jnp.sum(jnp.where(eqm, C, 0), axis=1, keepdims=True)
        s_at = jnp.sum(jnp.where(eqm, S, 0.0), axis=1, keepdims=True)
        r = (need - cb2).astype(jnp.float32)
        est = sbel + sb2 + r * s_at / jnp.maximum(c_at, 1).astype(jnp.float32)
        means = est / k_c.astype(jnp.float32)
        diff = jnp.sum(means)
        loss = jnp.maximum(_MARGIN - diff, 0.0)
        empty = jnp.sum(jnp.where(n_c == 0, 1, 0)) > 0
        loss = jnp.where(empty, 0.0, loss)
        loss_ref[...] = jnp.broadcast_to(loss, (1, 1))

    return pl.pallas_call(
        body,
        out_shape=jax.ShapeDtypeStruct((1, 1), jnp.float32),
        interpret=interpret,
    )


_sc_pass1_cached = functools.cache(_make_sc_pass1)
_sc_pass2_cached = functools.cache(_make_sc_pass2)
_tc_sel = _make_tc_select()
_tc_fin = _make_tc_final()


@jax.jit
def kernel(logits, labels):
    assert logits.shape[0] % (_NW * _CH) == 0
    cnt1 = _sc_pass1_cached()(logits, labels)
    pr1, cb1, kk1, nn1 = _tc_sel(cnt1)
    cnt2, sum2, bel = _sc_pass2_cached()(logits, labels, pr1)
    loss = _tc_fin(cnt2, sum2, bel, cb1, kk1, nn1)
    return loss.reshape(())
